# Initial kernel scaffold; baseline (speedup 1.0000x reference)
import jax
import jax.numpy as jnp
from jax.experimental import pallas as pl


def kernel(edge_exports, edge_imports, target_country_idx, target_product_idx, params):
    raise NotImplementedError("write your pallas kernel here")


# speedup vs baseline: 81.7057x; 81.7057x over previous
import functools
import math

import jax
import jax.numpy as jnp
from jax import lax
from jax.experimental import pallas as pl
from jax.experimental.pallas import tpu as pltpu
from jax.experimental.pallas import tpu_sc as plsc

NCN = 256
NPN = 8192
HID = 128
NHEAD = 4
DHD = HID // NHEAD
TT = 8
NE = 100000
NPAIR = 4096
F32 = jnp.float32
SCALE = 1.0 / math.sqrt(DHD)

_HIGH = jax.lax.Precision.HIGHEST

NW = 32
SLICE = 65536
CEDGE = 10000


def _mm(a, b):
    return lax.dot_general(a, b, (((1,), (0,)), ((), ())),
                           preferred_element_type=F32, precision=_HIGH)


def _mm_t(a, b):
    return lax.dot_general(a, b, (((1,), (1,)), ((), ())),
                           preferred_element_type=F32, precision=_HIGH)


def _flat_call(ee, ei):
    def body(ee_ref, ei_ref, fe_ref, fi_ref):
        fe_ref[0] = ee_ref[0, 1:2, :] * NCN + ee_ref[0, 0:1, :]
        fi_ref[0] = ei_ref[0, 1:2, :] * NPN + ei_ref[0, 0:1, :]
    fe, fi = pl.pallas_call(
        body, grid=(TT,),
        in_specs=[pl.BlockSpec((1, 2, NE), lambda t: (t, 0, 0))] * 2,
        out_specs=[pl.BlockSpec((1, 1, NE), lambda t: (t, 0, 0))] * 2,
        out_shape=[jax.ShapeDtypeStruct((TT, 1, NE), jnp.int32)] * 2,
    )(ee, ei)
    return fe.reshape(TT, NE), fi.reshape(TT, NE)


def _counts_call(flat_e, flat_i, zrow):
    mesh = plsc.VectorSubcoreMesh(core_axis_name="c", subcore_axis_name="s")

    @functools.partial(
        pl.kernel, mesh=mesh,
        compiler_params=pltpu.CompilerParams(needs_layout_passes=False),
        out_type=[jax.ShapeDtypeStruct((TT * NW * SLICE,), F32)] * 2,
        scratch_types=[pltpu.VMEM((CEDGE,), jnp.int32),
                       pltpu.VMEM((SLICE,), F32)],
    )
    def k(fe_hbm, fi_hbm, z_hbm, ce_hbm, ci_hbm, idx_v, acc_v):
        wid = lax.axis_index("s") * 2 + lax.axis_index("c")
        base = wid * SLICE
        ones = jnp.full((16,), 1.0, F32)
        for src_hbm, dst_hbm in ((fe_hbm, ce_hbm), (fi_hbm, ci_hbm)):
            def per_t(t, _):
                pltpu.sync_copy(z_hbm, acc_v)
                def per_chunk(c, _):
                    off = pl.multiple_of(t * NE + c * CEDGE, 8)
                    pltpu.sync_copy(src_hbm.at[pl.ds(off, CEDGE)], idx_v)
                    def per_vec(i, _):
                        fv = idx_v[pl.ds(i * 16, 16)]
                        rel = fv - base
                        m = (rel >= 0) & (rel < SLICE)
                        rel = jnp.where(m, rel, 0)
                        plsc.addupdate_scatter(acc_v, [rel], ones, mask=m)
                        return 0
                    return lax.fori_loop(0, CEDGE // 16, per_vec, 0)
                lax.fori_loop(0, NE // CEDGE, per_chunk, 0)
                ooff = pl.multiple_of((t * NW + wid) * SLICE, 8)
                pltpu.sync_copy(acc_v, dst_hbm.at[pl.ds(ooff, SLICE)])
                return 0
            lax.fori_loop(0, TT, per_t, 0)

    return k(flat_e.reshape(TT * NE), flat_i.reshape(TT * NE), zrow)


def _proj_call(x, wqT, bq, wkT, bk, ak, wvT, bv, av, bn):
    ttx, n, _ = x.shape
    grid = (ttx, n // bn)

    def body(x_ref, wq_ref, bq_ref, wk_ref, bk_ref, ak_ref, wv_ref, bv_ref,
             av_ref, q_ref, kr_ref, vr_ref):
        xb = x_ref[0]
        q_ref[0] = _mm(xb, wq_ref[...]) + bq_ref[...]
        kb = _mm(xb, wk_ref[...]) + bk_ref[...]
        kr_ref[0] = _mm(kb, ak_ref[...])
        vb = _mm(xb, wv_ref[...]) + bv_ref[...]
        vr_ref[0] = _mm(vb, av_ref[...])

    wspec = pl.BlockSpec((HID, HID), lambda t, b: (0, 0))
    bspec = pl.BlockSpec((1, HID), lambda t, b: (0, 0))
    xspec = pl.BlockSpec((1, bn, HID), lambda t, b: (t, b, 0))
    return pl.pallas_call(
        body, grid=grid,
        in_specs=[xspec, wspec, bspec, wspec, bspec, wspec, wspec, bspec, wspec],
        out_specs=[xspec] * 3,
        out_shape=[jax.ShapeDtypeStruct((ttx, n, HID), F32)] * 3,
    )(x, wqT, bq, wkT, bk, ak, wvT, bv, av)


def _msg_call(qd, krs, vrs, C, xin, owT, ob, skip, bd, cs):
    tq = qd.shape[0]
    tx = xin.shape[0]
    nd, ns = C.shape[1], C.shape[2]
    grid = (TT, nd // bd, ns // cs)
    nch = ns // cs

    dmap = ((lambda t, b, c: (t, b, 0)) if tq > 1 else
            (lambda t, b, c: (0, b, 0)))
    smap = ((lambda t, b, c: (t, c, 0)) if tq > 1 else
            (lambda t, b, c: (0, c, 0)))
    xmap = ((lambda t, b, c: (t, b, 0)) if tx > 1 else
            (lambda t, b, c: (0, b, 0)))

    def body(q_ref, kr_ref, vr_ref, c_ref, x_ref, ow_ref, ob_ref, sk_ref,
             out_ref, acc_ref, den_ref):
        ci = pl.program_id(2)

        @pl.when(ci == 0)
        def _zero():
            acc_ref[...] = jnp.zeros_like(acc_ref)
            den_ref[...] = jnp.zeros_like(den_ref)

        q = q_ref[0]
        kr = kr_ref[0]
        vr = vr_ref[0]
        cb = c_ref[0]
        for h in range(NHEAD):
            sl = slice(h * DHD, (h + 1) * DHD)
            al = _mm_t(q[:, sl], kr[:, sl])
            w = jnp.exp(al) * cb
            den_ref[:, h:h + 1] += jnp.sum(w, axis=1, keepdims=True)
            acc_ref[:, sl] += _mm(w, vr[:, sl])

        @pl.when(ci == nch - 1)
        def _fin():
            agg = acc_ref[...]
            den = den_ref[...]
            parts = [agg[:, h * DHD:(h + 1) * DHD] / (den[:, h:h + 1] + 1e-16)
                     for h in range(NHEAD)]
            nag = jnp.concatenate(parts, axis=1)
            o = _mm(jax.nn.gelu(nag), ow_ref[...]) + ob_ref[...]
            bt = jax.nn.sigmoid(sk_ref[0, 0])
            out_ref[0] = bt * o + (1.0 - bt) * x_ref[0]

    return pl.pallas_call(
        body, grid=grid,
        in_specs=[
            pl.BlockSpec((1, bd, HID), dmap),
            pl.BlockSpec((1, cs, HID), smap),
            pl.BlockSpec((1, cs, HID), smap),
            pl.BlockSpec((1, bd, cs), lambda t, b, c: (t, b, c)),
            pl.BlockSpec((1, bd, HID), xmap),
            pl.BlockSpec((HID, HID), lambda t, b, c: (0, 0)),
            pl.BlockSpec((1, HID), lambda t, b, c: (0, 0)),
            pl.BlockSpec((1, 1), lambda t, b, c: (0, 0)),
        ],
        out_specs=pl.BlockSpec((1, bd, HID), lambda t, b, c: (t, b, 0)),
        out_shape=jax.ShapeDtypeStruct((TT, nd, HID), F32),
        scratch_shapes=[pltpu.VMEM((bd, HID), F32),
                        pltpu.VMEM((bd, NHEAD), F32)],
    )(qd, krs, vrs, C, xin, owT, ob, skip)


def _gru_call(seq, wihT, whhT, bih, bhh, br):
    nr = seq.shape[0]
    grid = (nr // br,)

    def body(s_ref, wih_ref, whh_ref, bih_ref, bhh_ref, out_ref):
        h = jnp.zeros((br, HID), F32)
        for t in range(TT):
            x = s_ref[:, t, :]
            gi = _mm(x, wih_ref[...]) + bih_ref[...]
            gh = _mm(h, whh_ref[...]) + bhh_ref[...]
            r = jax.nn.sigmoid(gi[:, :HID] + gh[:, :HID])
            z = jax.nn.sigmoid(gi[:, HID:2 * HID] + gh[:, HID:2 * HID])
            n = jnp.tanh(gi[:, 2 * HID:] + r * gh[:, 2 * HID:])
            h = (1.0 - z) * n + z * h
        out_ref[...] = h

    return pl.pallas_call(
        body, grid=grid,
        in_specs=[pl.BlockSpec((br, TT, HID), lambda b: (b, 0, 0)),
                  pl.BlockSpec((HID, 3 * HID), lambda b: (0, 0)),
                  pl.BlockSpec((HID, 3 * HID), lambda b: (0, 0)),
                  pl.BlockSpec((1, 3 * HID), lambda b: (0, 0)),
                  pl.BlockSpec((1, 3 * HID), lambda b: (0, 0))],
        out_specs=pl.BlockSpec((br, HID), lambda b: (b, 0)),
        out_shape=jax.ShapeDtypeStruct((nr, HID), F32),
    )(seq, wihT, whhT, bih, bhh)


def _pair_gather(cemb, pemb, tci, tpi):
    mesh = plsc.VectorSubcoreMesh(core_axis_name="c", subcore_axis_name="s")
    bw = NPAIR // NW

    @functools.partial(
        pl.kernel, mesh=mesh,
        compiler_params=pltpu.CompilerParams(needs_layout_passes=False),
        out_type=[jax.ShapeDtypeStruct((NPAIR, HID), F32)] * 2,
        scratch_types=[pltpu.VMEM((bw,), jnp.int32),
                       pltpu.VMEM((bw, HID), F32),
                       pltpu.SemaphoreType.DMA],
    )
    def k(c_hbm, p_hbm, tci_hbm, tpi_hbm, oc_hbm, op_hbm, idx_v, rows_v, sem):
        wid = lax.axis_index("s") * 2 + lax.axis_index("c")
        base = pl.multiple_of(wid * bw, 8)
        pltpu.sync_copy(tci_hbm.at[pl.ds(base, bw)], idx_v)
        pltpu.async_copy(c_hbm.at[idx_v], rows_v, sem).wait()
        pltpu.sync_copy(rows_v, oc_hbm.at[pl.ds(base, bw)])
        pltpu.sync_copy(tpi_hbm.at[pl.ds(base, bw)], idx_v)
        pltpu.async_copy(p_hbm.at[idx_v], rows_v, sem).wait()
        pltpu.sync_copy(rows_v, op_hbm.at[pl.ds(base, bw)])

    return k(cemb, pemb, tci, tpi)


def _head_call(pair, w1T, b1, w2p, b2p, br):
    grid = (NPAIR // br,)

    def body(p_ref, w1_ref, b1_ref, w2_ref, b2_ref, out_ref):
        hdn = jnp.maximum(_mm(p_ref[...], w1_ref[...]) + b1_ref[...], 0.0)
        lg = _mm(hdn, w2_ref[...]) + b2_ref[...]
        out_ref[...] = jax.nn.sigmoid(lg)

    return pl.pallas_call(
        body, grid=grid,
        in_specs=[pl.BlockSpec((br, 2 * HID), lambda b: (b, 0)),
                  pl.BlockSpec((2 * HID, HID), lambda b: (0, 0)),
                  pl.BlockSpec((1, HID), lambda b: (0, 0)),
                  pl.BlockSpec((HID, HID), lambda b: (0, 0)),
                  pl.BlockSpec((1, HID), lambda b: (0, 0))],
        out_specs=pl.BlockSpec((br, HID), lambda b: (b, 0)),
        out_shape=jax.ShapeDtypeStruct((NPAIR, HID), F32),
    )(pair, w1T, b1, w2p, b2p)


def _fold(lp):
    bd = jax.scipy.linalg.block_diag
    f = {}
    f['akc'] = bd(*[lp['a_exp'][h] * (lp['p_exp'][h] * SCALE)
                    for h in range(NHEAD)])
    f['avc'] = bd(*[lp['m_exp'][h] for h in range(NHEAD)])
    f['akp'] = bd(*[lp['a_imp'][h] * (lp['p_imp'][h] * SCALE)
                    for h in range(NHEAD)])
    f['avp'] = bd(*[lp['m_imp'][h] for h in range(NHEAD)])
    for nt in ('c', 'p'):
        for pr in ('q', 'k', 'v', 'o'):
            f[pr + nt + 'T'] = lp[pr + '_' + nt + '_w'].T
            f['b' + pr + nt] = lp[pr + '_' + nt + '_b'][None]
        f['sk' + nt] = lp['skip_' + nt].reshape(1, 1)
    return f


def kernel(edge_exports, edge_imports, target_country_idx, target_product_idx,
           params):
    p = params
    flat_e, flat_i = _flat_call(edge_exports.astype(jnp.int32),
                                edge_imports.astype(jnp.int32))
    zrow = jnp.zeros((SLICE,), F32)
    ce_raw, ci_raw = _counts_call(flat_e, flat_i, zrow)
    c_exp = ce_raw.reshape(TT, NPN, NCN)
    c_imp = ci_raw.reshape(TT, NCN, NPN)

    x_c = p['country_emb'][None]
    x_p = p['product_emb'][None]
    for lp in p['layers']:
        f = _fold(lp)
        q_c, kr_c, vr_c = _proj_call(x_c, f['qcT'], f['bqc'], f['kcT'],
                                     f['bkc'], f['akc'], f['vcT'], f['bvc'],
                                     f['avc'], bn=NCN)
        q_p, kr_p, vr_p = _proj_call(x_p, f['qpT'], f['bqp'], f['kpT'],
                                     f['bkp'], f['akp'], f['vpT'], f['bvp'],
                                     f['avp'], bn=512)
        x_p_new = _msg_call(q_p, kr_c, vr_c, c_exp, x_p, f['opT'], f['bop'],
                            f['skp'], bd=512, cs=NCN)
        x_c_new = _msg_call(q_c, kr_p, vr_p, c_imp, x_c, f['ocT'], f['boc'],
                            f['skc'], bd=NCN, cs=2048)
        x_c, x_p = x_c_new, x_p_new

    cseq = jnp.transpose(x_c, (1, 0, 2))
    pseq = jnp.transpose(x_p, (1, 0, 2))
    seq = jnp.concatenate([cseq, pseq], axis=0)
    g = p['gru']
    hfin = _gru_call(seq, g['w_ih'].T, g['w_hh'].T, g['b_ih'][None],
                     g['b_hh'][None], br=768)
    c_emb, p_emb = hfin[:NCN], hfin[NCN:]

    crow, prow = _pair_gather(c_emb, p_emb, target_country_idx.astype(jnp.int32),
                              target_product_idx.astype(jnp.int32))
    pair = jnp.concatenate([crow, prow], axis=1)
    w2p = jnp.zeros((HID, HID), F32).at[:, 0].set(p['lp_w2'][0])
    b2p = jnp.broadcast_to(p['lp_b2'][None], (1, HID))
    out = _head_call(pair, p['lp_w1'].T, p['lp_b1'][None], w2p, b2p, br=1024)
    return out[:, :1]


# speedup vs baseline: 128.8887x; 1.5775x over previous
import functools
import math

import jax
import jax.numpy as jnp
from jax import lax
from jax.experimental import pallas as pl
from jax.experimental.pallas import tpu as pltpu
from jax.experimental.pallas import tpu_sc as plsc

NCN = 256
NPN = 8192
HID = 128
NHEAD = 4
DHD = HID // NHEAD
TT = 8
NE = 100000
NPAIR = 4096
F32 = jnp.float32
SCALE = 1.0 / math.sqrt(DHD)

_HIGH = jax.lax.Precision.DEFAULT

NW = 32
SLICE = 65536
CEDGE = 10000


def _mm(a, b):
    return lax.dot_general(a, b, (((1,), (0,)), ((), ())),
                           preferred_element_type=F32, precision=_HIGH)


def _mm_t(a, b):
    return lax.dot_general(a, b, (((1,), (1,)), ((), ())),
                           preferred_element_type=F32, precision=_HIGH)


def _flat_call(ee, ei):
    def body(ee_ref, ei_ref, fe_ref, fi_ref):
        fe_ref[0] = ee_ref[0, 1:2, :] * NCN + ee_ref[0, 0:1, :]
        fi_ref[0] = ei_ref[0, 1:2, :] * NPN + ei_ref[0, 0:1, :]
    fe, fi = pl.pallas_call(
        body, grid=(TT,),
        in_specs=[pl.BlockSpec((1, 2, NE), lambda t: (t, 0, 0))] * 2,
        out_specs=[pl.BlockSpec((1, 1, NE), lambda t: (t, 0, 0))] * 2,
        out_shape=[jax.ShapeDtypeStruct((TT, 1, NE), jnp.int32)] * 2,
    )(ee, ei)
    return fe.reshape(TT, NE), fi.reshape(TT, NE)


def _counts_call(flat_e, flat_i, zrow):
    mesh = plsc.VectorSubcoreMesh(core_axis_name="c", subcore_axis_name="s")

    @functools.partial(
        pl.kernel, mesh=mesh,
        compiler_params=pltpu.CompilerParams(needs_layout_passes=False),
        out_type=[jax.ShapeDtypeStruct((TT * NW * SLICE,), F32)] * 2,
        scratch_types=[pltpu.VMEM((CEDGE,), jnp.int32),
                       pltpu.VMEM((SLICE,), F32)],
    )
    def k(fe_hbm, fi_hbm, z_hbm, ce_hbm, ci_hbm, idx_v, acc_v):
        wid = lax.axis_index("s") * 2 + lax.axis_index("c")
        base = wid * SLICE
        ones = jnp.full((16,), 1.0, F32)
        for src_hbm, dst_hbm in ((fe_hbm, ce_hbm), (fi_hbm, ci_hbm)):
            def per_t(t, _):
                pltpu.sync_copy(z_hbm, acc_v)
                def per_chunk(c, _):
                    off = pl.multiple_of(t * NE + c * CEDGE, 8)
                    pltpu.sync_copy(src_hbm.at[pl.ds(off, CEDGE)], idx_v)
                    def per_vec(i, _):
                        fv = idx_v[pl.ds(i * 16, 16)]
                        rel = fv - base
                        m = (rel >= 0) & (rel < SLICE)
                        rel = jnp.where(m, rel, 0)
                        plsc.addupdate_scatter(acc_v, [rel], ones, mask=m)
                        return 0
                    return lax.fori_loop(0, CEDGE // 16, per_vec, 0)
                lax.fori_loop(0, NE // CEDGE, per_chunk, 0)
                ooff = pl.multiple_of((t * NW + wid) * SLICE, 8)
                pltpu.sync_copy(acc_v, dst_hbm.at[pl.ds(ooff, SLICE)])
                return 0
            lax.fori_loop(0, TT, per_t, 0)

    return k(flat_e.reshape(TT * NE), flat_i.reshape(TT * NE), zrow)


def _proj_call(x, wqT, bq, wkT, bk, ak, wvT, bv, av, bn):
    ttx, n, _ = x.shape
    grid = (ttx, n // bn)

    def body(x_ref, wq_ref, bq_ref, wk_ref, bk_ref, ak_ref, wv_ref, bv_ref,
             av_ref, q_ref, kr_ref, vr_ref):
        xb = x_ref[0]
        q_ref[0] = _mm(xb, wq_ref[...]) + bq_ref[...]
        kb = _mm(xb, wk_ref[...]) + bk_ref[...]
        kr_ref[0] = _mm(kb, ak_ref[...])
        vb = _mm(xb, wv_ref[...]) + bv_ref[...]
        vr_ref[0] = _mm(vb, av_ref[...])

    wspec = pl.BlockSpec((HID, HID), lambda t, b: (0, 0))
    bspec = pl.BlockSpec((1, HID), lambda t, b: (0, 0))
    xspec = pl.BlockSpec((1, bn, HID), lambda t, b: (t, b, 0))
    return pl.pallas_call(
        body, grid=grid,
        in_specs=[xspec, wspec, bspec, wspec, bspec, wspec, wspec, bspec, wspec],
        out_specs=[xspec] * 3,
        out_shape=[jax.ShapeDtypeStruct((ttx, n, HID), F32)] * 3,
    )(x, wqT, bq, wkT, bk, ak, wvT, bv, av)


def _msg_call(qd, krs, vrs, C, xin, owT, ob, skip, bd, cs):
    tq = qd.shape[0]
    tx = xin.shape[0]
    nd, ns = C.shape[1], C.shape[2]
    grid = (TT, nd // bd, ns // cs)
    nch = ns // cs

    dmap = ((lambda t, b, c: (t, b, 0)) if tq > 1 else
            (lambda t, b, c: (0, b, 0)))
    smap = ((lambda t, b, c: (t, c, 0)) if tq > 1 else
            (lambda t, b, c: (0, c, 0)))
    xmap = ((lambda t, b, c: (t, b, 0)) if tx > 1 else
            (lambda t, b, c: (0, b, 0)))

    def body(q_ref, kr_ref, vr_ref, c_ref, x_ref, ow_ref, ob_ref, sk_ref,
             out_ref, acc_ref, den_ref):
        ci = pl.program_id(2)

        @pl.when(ci == 0)
        def _zero():
            acc_ref[...] = jnp.zeros_like(acc_ref)
            den_ref[...] = jnp.zeros_like(den_ref)

        q = q_ref[0]
        kr = kr_ref[0]
        vr = vr_ref[0]
        cb = c_ref[0]
        for h in range(NHEAD):
            sl = slice(h * DHD, (h + 1) * DHD)
            al = _mm_t(q[:, sl], kr[:, sl])
            w = jnp.exp(al) * cb
            den_ref[:, h:h + 1] += jnp.sum(w, axis=1, keepdims=True)
            acc_ref[:, sl] += _mm(w, vr[:, sl])

        @pl.when(ci == nch - 1)
        def _fin():
            agg = acc_ref[...]
            den = den_ref[...]
            parts = [agg[:, h * DHD:(h + 1) * DHD] / (den[:, h:h + 1] + 1e-16)
                     for h in range(NHEAD)]
            nag = jnp.concatenate(parts, axis=1)
            o = _mm(jax.nn.gelu(nag), ow_ref[...]) + ob_ref[...]
            bt = jax.nn.sigmoid(sk_ref[0, 0])
            out_ref[0] = bt * o + (1.0 - bt) * x_ref[0]

    return pl.pallas_call(
        body, grid=grid,
        in_specs=[
            pl.BlockSpec((1, bd, HID), dmap),
            pl.BlockSpec((1, cs, HID), smap),
            pl.BlockSpec((1, cs, HID), smap),
            pl.BlockSpec((1, bd, cs), lambda t, b, c: (t, b, c)),
            pl.BlockSpec((1, bd, HID), xmap),
            pl.BlockSpec((HID, HID), lambda t, b, c: (0, 0)),
            pl.BlockSpec((1, HID), lambda t, b, c: (0, 0)),
            pl.BlockSpec((1, 1), lambda t, b, c: (0, 0)),
        ],
        out_specs=pl.BlockSpec((1, bd, HID), lambda t, b, c: (t, b, 0)),
        out_shape=jax.ShapeDtypeStruct((TT, nd, HID), F32),
        scratch_shapes=[pltpu.VMEM((bd, HID), F32),
                        pltpu.VMEM((bd, NHEAD), F32)],
    )(qd, krs, vrs, C, xin, owT, ob, skip)


def _gru_call(seq, wihT, whhT, bih, bhh, br):
    nr = seq.shape[0]
    grid = (nr // br,)

    def body(s_ref, wih_ref, whh_ref, bih_ref, bhh_ref, out_ref):
        h = jnp.zeros((br, HID), F32)
        for t in range(TT):
            x = s_ref[:, t, :]
            gi = _mm(x, wih_ref[...]) + bih_ref[...]
            gh = _mm(h, whh_ref[...]) + bhh_ref[...]
            r = jax.nn.sigmoid(gi[:, :HID] + gh[:, :HID])
            z = jax.nn.sigmoid(gi[:, HID:2 * HID] + gh[:, HID:2 * HID])
            n = jnp.tanh(gi[:, 2 * HID:] + r * gh[:, 2 * HID:])
            h = (1.0 - z) * n + z * h
        out_ref[...] = h

    return pl.pallas_call(
        body, grid=grid,
        in_specs=[pl.BlockSpec((br, TT, HID), lambda b: (b, 0, 0)),
                  pl.BlockSpec((HID, 3 * HID), lambda b: (0, 0)),
                  pl.BlockSpec((HID, 3 * HID), lambda b: (0, 0)),
                  pl.BlockSpec((1, 3 * HID), lambda b: (0, 0)),
                  pl.BlockSpec((1, 3 * HID), lambda b: (0, 0))],
        out_specs=pl.BlockSpec((br, HID), lambda b: (b, 0)),
        out_shape=jax.ShapeDtypeStruct((nr, HID), F32),
    )(seq, wihT, whhT, bih, bhh)


def _pair_gather(cemb, pemb, tci, tpi):
    mesh = plsc.VectorSubcoreMesh(core_axis_name="c", subcore_axis_name="s")
    bw = NPAIR // NW

    @functools.partial(
        pl.kernel, mesh=mesh,
        compiler_params=pltpu.CompilerParams(needs_layout_passes=False),
        out_type=[jax.ShapeDtypeStruct((NPAIR, HID), F32)] * 2,
        scratch_types=[pltpu.VMEM((bw,), jnp.int32),
                       pltpu.VMEM((bw, HID), F32),
                       pltpu.SemaphoreType.DMA],
    )
    def k(c_hbm, p_hbm, tci_hbm, tpi_hbm, oc_hbm, op_hbm, idx_v, rows_v, sem):
        wid = lax.axis_index("s") * 2 + lax.axis_index("c")
        base = pl.multiple_of(wid * bw, 8)
        pltpu.sync_copy(tci_hbm.at[pl.ds(base, bw)], idx_v)
        pltpu.async_copy(c_hbm.at[idx_v], rows_v, sem).wait()
        pltpu.sync_copy(rows_v, oc_hbm.at[pl.ds(base, bw)])
        pltpu.sync_copy(tpi_hbm.at[pl.ds(base, bw)], idx_v)
        pltpu.async_copy(p_hbm.at[idx_v], rows_v, sem).wait()
        pltpu.sync_copy(rows_v, op_hbm.at[pl.ds(base, bw)])

    return k(cemb, pemb, tci, tpi)


def _head_call(pair, w1T, b1, w2p, b2p, br):
    grid = (NPAIR // br,)

    def body(p_ref, w1_ref, b1_ref, w2_ref, b2_ref, out_ref):
        hdn = jnp.maximum(_mm(p_ref[...], w1_ref[...]) + b1_ref[...], 0.0)
        lg = _mm(hdn, w2_ref[...]) + b2_ref[...]
        out_ref[...] = jax.nn.sigmoid(lg)

    return pl.pallas_call(
        body, grid=grid,
        in_specs=[pl.BlockSpec((br, 2 * HID), lambda b: (b, 0)),
                  pl.BlockSpec((2 * HID, HID), lambda b: (0, 0)),
                  pl.BlockSpec((1, HID), lambda b: (0, 0)),
                  pl.BlockSpec((HID, HID), lambda b: (0, 0)),
                  pl.BlockSpec((1, HID), lambda b: (0, 0))],
        out_specs=pl.BlockSpec((br, HID), lambda b: (b, 0)),
        out_shape=jax.ShapeDtypeStruct((NPAIR, HID), F32),
    )(pair, w1T, b1, w2p, b2p)


def _fold(lp):
    bd = jax.scipy.linalg.block_diag
    f = {}
    f['akc'] = bd(*[lp['a_exp'][h] * (lp['p_exp'][h] * SCALE)
                    for h in range(NHEAD)])
    f['avc'] = bd(*[lp['m_exp'][h] for h in range(NHEAD)])
    f['akp'] = bd(*[lp['a_imp'][h] * (lp['p_imp'][h] * SCALE)
                    for h in range(NHEAD)])
    f['avp'] = bd(*[lp['m_imp'][h] for h in range(NHEAD)])
    for nt in ('c', 'p'):
        for pr in ('q', 'k', 'v', 'o'):
            f[pr + nt + 'T'] = lp[pr + '_' + nt + '_w'].T
            f['b' + pr + nt] = lp[pr + '_' + nt + '_b'][None]
        f['sk' + nt] = lp['skip_' + nt].reshape(1, 1)
    return f


def kernel(edge_exports, edge_imports, target_country_idx, target_product_idx,
           params):
    p = params
    flat_e, flat_i = _flat_call(edge_exports.astype(jnp.int32),
                                edge_imports.astype(jnp.int32))
    zrow = jnp.zeros((SLICE,), F32)
    ce_raw, ci_raw = _counts_call(flat_e, flat_i, zrow)
    c_exp = ce_raw.reshape(TT, NPN, NCN)
    c_imp = ci_raw.reshape(TT, NCN, NPN)

    x_c = p['country_emb'][None]
    x_p = p['product_emb'][None]
    for lp in p['layers']:
        f = _fold(lp)
        q_c, kr_c, vr_c = _proj_call(x_c, f['qcT'], f['bqc'], f['kcT'],
                                     f['bkc'], f['akc'], f['vcT'], f['bvc'],
                                     f['avc'], bn=NCN)
        q_p, kr_p, vr_p = _proj_call(x_p, f['qpT'], f['bqp'], f['kpT'],
                                     f['bkp'], f['akp'], f['vpT'], f['bvp'],
                                     f['avp'], bn=512)
        x_p_new = _msg_call(q_p, kr_c, vr_c, c_exp, x_p, f['opT'], f['bop'],
                            f['skp'], bd=512, cs=NCN)
        x_c_new = _msg_call(q_c, kr_p, vr_p, c_imp, x_c, f['ocT'], f['boc'],
                            f['skc'], bd=NCN, cs=2048)
        x_c, x_p = x_c_new, x_p_new

    cseq = jnp.transpose(x_c, (1, 0, 2))
    pseq = jnp.transpose(x_p, (1, 0, 2))
    seq = jnp.concatenate([cseq, pseq], axis=0)
    g = p['gru']
    hfin = _gru_call(seq, g['w_ih'].T, g['w_hh'].T, g['b_ih'][None],
                     g['b_hh'][None], br=768)
    c_emb, p_emb = hfin[:NCN], hfin[NCN:]

    crow, prow = _pair_gather(c_emb, p_emb, target_country_idx.astype(jnp.int32),
                              target_product_idx.astype(jnp.int32))
    pair = jnp.concatenate([crow, prow], axis=1)
    w2p = jnp.zeros((HID, HID), F32).at[:, 0].set(p['lp_w2'][0])
    b2p = jnp.broadcast_to(p['lp_b2'][None], (1, HID))
    out = _head_call(pair, p['lp_w1'].T, p['lp_b1'][None], w2p, b2p, br=1024)
    return out[:, :1]


# speedup vs baseline: 141.0841x; 1.0946x over previous
import functools
import math

import jax
import jax.numpy as jnp
from jax import lax
from jax.experimental import pallas as pl
from jax.experimental.pallas import tpu as pltpu
from jax.experimental.pallas import tpu_sc as plsc

NCN = 256
NPN = 8192
HID = 128
NHEAD = 4
DHD = HID // NHEAD
TT = 8
NE = 100000
NPAIR = 4096
F32 = jnp.float32
SCALE = 1.0 / math.sqrt(DHD)

_HIGH = jax.lax.Precision.DEFAULT

NW = 32
SLICE = 65536
CEDGE = 10000


def _mm(a, b):
    return lax.dot_general(a, b, (((1,), (0,)), ((), ())),
                           preferred_element_type=F32, precision=_HIGH)


def _mm_t(a, b):
    return lax.dot_general(a, b, (((1,), (1,)), ((), ())),
                           preferred_element_type=F32, precision=_HIGH)


def _flat_call(ee, ei):
    def body(ee_ref, ei_ref, fe_ref, fi_ref):
        fe_ref[0] = ee_ref[0, 1:2, :] * NCN + ee_ref[0, 0:1, :]
        fi_ref[0] = ei_ref[0, 1:2, :] * NPN + ei_ref[0, 0:1, :]
    fe, fi = pl.pallas_call(
        body, grid=(TT,),
        in_specs=[pl.BlockSpec((1, 2, NE), lambda t: (t, 0, 0))] * 2,
        out_specs=[pl.BlockSpec((1, 1, NE), lambda t: (t, 0, 0))] * 2,
        out_shape=[jax.ShapeDtypeStruct((TT, 1, NE), jnp.int32)] * 2,
    )(ee, ei)
    return fe.reshape(TT, NE), fi.reshape(TT, NE)


def _counts_call(flat_e, flat_i, zrow):
    mesh = plsc.VectorSubcoreMesh(core_axis_name="c", subcore_axis_name="s")

    nchunk = NE // CEDGE
    unroll = 5

    @functools.partial(
        pl.kernel, mesh=mesh,
        compiler_params=pltpu.CompilerParams(needs_layout_passes=False),
        out_type=[jax.ShapeDtypeStruct((TT * NW * SLICE,), F32)] * 2,
        scratch_types=[pltpu.VMEM((CEDGE,), jnp.int32),
                       pltpu.VMEM((CEDGE,), jnp.int32),
                       pltpu.VMEM((SLICE,), F32),
                       pltpu.SemaphoreType.DMA,
                       pltpu.SemaphoreType.DMA,
                       pltpu.SemaphoreType.DMA],
    )
    def k(fe_hbm, fi_hbm, z_hbm, ce_hbm, ci_hbm, buf0, buf1, acc_v,
          sem0, sem1, zsem):
        wid = lax.axis_index("s") * 2 + lax.axis_index("c")
        base = wid * SLICE
        ones = jnp.full((16,), 1.0, F32)
        lim = jnp.uint32(SLICE)

        def scatter_chunk(buf):
            def per_vec(i, _):
                for u in range(unroll):
                    fv = buf[pl.ds((i * unroll + u) * 16, 16)]
                    rel = fv - base
                    m = plsc.bitcast(rel, jnp.uint32) < lim
                    rel = jnp.where(m, rel, 0)
                    plsc.addupdate_scatter(acc_v, [rel], ones, mask=m)
                return 0
            lax.fori_loop(0, (CEDGE // 16) // unroll, per_vec, 0)

        def chunk_at(src_hbm, t, c):
            off = pl.multiple_of(t * NE + c * CEDGE, 8)
            return src_hbm.at[pl.ds(off, CEDGE)]

        for src_hbm, dst_hbm in ((fe_hbm, ce_hbm), (fi_hbm, ci_hbm)):
            def per_t(t, _):
                zcp = pltpu.async_copy(z_hbm, acc_v, zsem)
                cp0 = pltpu.async_copy(chunk_at(src_hbm, t, 0), buf0, sem0)
                zcp.wait()
                cp0.wait()

                def per_g(g, _):
                    cpa = pltpu.async_copy(chunk_at(src_hbm, t, 2 * g + 1),
                                           buf1, sem1)
                    scatter_chunk(buf0)
                    cpa.wait()
                    cpb = pltpu.async_copy(chunk_at(src_hbm, t, 2 * g + 2),
                                           buf0, sem0)
                    scatter_chunk(buf1)
                    cpb.wait()
                    return 0
                lax.fori_loop(0, nchunk // 2 - 1, per_g, 0)
                cpl = pltpu.async_copy(chunk_at(src_hbm, t, nchunk - 1),
                                       buf1, sem1)
                scatter_chunk(buf0)
                cpl.wait()
                scatter_chunk(buf1)
                ooff = pl.multiple_of((t * NW + wid) * SLICE, 8)
                pltpu.sync_copy(acc_v, dst_hbm.at[pl.ds(ooff, SLICE)])
                return 0
            lax.fori_loop(0, TT, per_t, 0)

    return k(flat_e.reshape(TT * NE), flat_i.reshape(TT * NE), zrow)


def _proj_call(x, wqT, bq, wkT, bk, ak, wvT, bv, av, bn):
    ttx, n, _ = x.shape
    grid = (ttx, n // bn)

    def body(x_ref, wq_ref, bq_ref, wk_ref, bk_ref, ak_ref, wv_ref, bv_ref,
             av_ref, q_ref, kr_ref, vr_ref):
        xb = x_ref[0]
        q_ref[0] = _mm(xb, wq_ref[...]) + bq_ref[...]
        kb = _mm(xb, wk_ref[...]) + bk_ref[...]
        kr_ref[0] = _mm(kb, ak_ref[...])
        vb = _mm(xb, wv_ref[...]) + bv_ref[...]
        vr_ref[0] = _mm(vb, av_ref[...])

    wspec = pl.BlockSpec((HID, HID), lambda t, b: (0, 0))
    bspec = pl.BlockSpec((1, HID), lambda t, b: (0, 0))
    xspec = pl.BlockSpec((1, bn, HID), lambda t, b: (t, b, 0))
    return pl.pallas_call(
        body, grid=grid,
        in_specs=[xspec, wspec, bspec, wspec, bspec, wspec, wspec, bspec, wspec],
        out_specs=[xspec] * 3,
        out_shape=[jax.ShapeDtypeStruct((ttx, n, HID), F32)] * 3,
    )(x, wqT, bq, wkT, bk, ak, wvT, bv, av)


def _msg_call(qd, krs, vrs, C, xin, owT, ob, skip, bd, cs):
    tq = qd.shape[0]
    tx = xin.shape[0]
    nd, ns = C.shape[1], C.shape[2]
    grid = (TT, nd // bd, ns // cs)
    nch = ns // cs

    dmap = ((lambda t, b, c: (t, b, 0)) if tq > 1 else
            (lambda t, b, c: (0, b, 0)))
    smap = ((lambda t, b, c: (t, c, 0)) if tq > 1 else
            (lambda t, b, c: (0, c, 0)))
    xmap = ((lambda t, b, c: (t, b, 0)) if tx > 1 else
            (lambda t, b, c: (0, b, 0)))

    def body(q_ref, kr_ref, vr_ref, c_ref, x_ref, ow_ref, ob_ref, sk_ref,
             out_ref, acc_ref, den_ref):
        ci = pl.program_id(2)

        @pl.when(ci == 0)
        def _zero():
            acc_ref[...] = jnp.zeros_like(acc_ref)
            den_ref[...] = jnp.zeros_like(den_ref)

        q = q_ref[0]
        kr = kr_ref[0]
        vr = vr_ref[0]
        cb = c_ref[0]
        for h in range(NHEAD):
            sl = slice(h * DHD, (h + 1) * DHD)
            al = _mm_t(q[:, sl], kr[:, sl])
            w = jnp.exp(al) * cb
            den_ref[:, h:h + 1] += jnp.sum(w, axis=1, keepdims=True)
            acc_ref[:, sl] += _mm(w, vr[:, sl])

        @pl.when(ci == nch - 1)
        def _fin():
            agg = acc_ref[...]
            den = den_ref[...]
            parts = [agg[:, h * DHD:(h + 1) * DHD] / (den[:, h:h + 1] + 1e-16)
                     for h in range(NHEAD)]
            nag = jnp.concatenate(parts, axis=1)
            o = _mm(jax.nn.gelu(nag), ow_ref[...]) + ob_ref[...]
            bt = jax.nn.sigmoid(sk_ref[0, 0])
            out_ref[0] = bt * o + (1.0 - bt) * x_ref[0]

    return pl.pallas_call(
        body, grid=grid,
        in_specs=[
            pl.BlockSpec((1, bd, HID), dmap),
            pl.BlockSpec((1, cs, HID), smap),
            pl.BlockSpec((1, cs, HID), smap),
            pl.BlockSpec((1, bd, cs), lambda t, b, c: (t, b, c)),
            pl.BlockSpec((1, bd, HID), xmap),
            pl.BlockSpec((HID, HID), lambda t, b, c: (0, 0)),
            pl.BlockSpec((1, HID), lambda t, b, c: (0, 0)),
            pl.BlockSpec((1, 1), lambda t, b, c: (0, 0)),
        ],
        out_specs=pl.BlockSpec((1, bd, HID), lambda t, b, c: (t, b, 0)),
        out_shape=jax.ShapeDtypeStruct((TT, nd, HID), F32),
        scratch_shapes=[pltpu.VMEM((bd, HID), F32),
                        pltpu.VMEM((bd, NHEAD), F32)],
    )(qd, krs, vrs, C, xin, owT, ob, skip)


def _gru_call(seq, wihT, whhT, bih, bhh, br):
    nr = seq.shape[0]
    grid = (nr // br,)

    def body(s_ref, wih_ref, whh_ref, bih_ref, bhh_ref, out_ref):
        h = jnp.zeros((br, HID), F32)
        for t in range(TT):
            x = s_ref[:, t, :]
            gi = _mm(x, wih_ref[...]) + bih_ref[...]
            gh = _mm(h, whh_ref[...]) + bhh_ref[...]
            r = jax.nn.sigmoid(gi[:, :HID] + gh[:, :HID])
            z = jax.nn.sigmoid(gi[:, HID:2 * HID] + gh[:, HID:2 * HID])
            n = jnp.tanh(gi[:, 2 * HID:] + r * gh[:, 2 * HID:])
            h = (1.0 - z) * n + z * h
        out_ref[...] = h

    return pl.pallas_call(
        body, grid=grid,
        in_specs=[pl.BlockSpec((br, TT, HID), lambda b: (b, 0, 0)),
                  pl.BlockSpec((HID, 3 * HID), lambda b: (0, 0)),
                  pl.BlockSpec((HID, 3 * HID), lambda b: (0, 0)),
                  pl.BlockSpec((1, 3 * HID), lambda b: (0, 0)),
                  pl.BlockSpec((1, 3 * HID), lambda b: (0, 0))],
        out_specs=pl.BlockSpec((br, HID), lambda b: (b, 0)),
        out_shape=jax.ShapeDtypeStruct((nr, HID), F32),
    )(seq, wihT, whhT, bih, bhh)


def _pair_gather(cemb, pemb, tci, tpi):
    mesh = plsc.VectorSubcoreMesh(core_axis_name="c", subcore_axis_name="s")
    bw = NPAIR // NW

    @functools.partial(
        pl.kernel, mesh=mesh,
        compiler_params=pltpu.CompilerParams(needs_layout_passes=False),
        out_type=[jax.ShapeDtypeStruct((NPAIR, HID), F32)] * 2,
        scratch_types=[pltpu.VMEM((bw,), jnp.int32),
                       pltpu.VMEM((bw, HID), F32),
                       pltpu.SemaphoreType.DMA],
    )
    def k(c_hbm, p_hbm, tci_hbm, tpi_hbm, oc_hbm, op_hbm, idx_v, rows_v, sem):
        wid = lax.axis_index("s") * 2 + lax.axis_index("c")
        base = pl.multiple_of(wid * bw, 8)
        pltpu.sync_copy(tci_hbm.at[pl.ds(base, bw)], idx_v)
        pltpu.async_copy(c_hbm.at[idx_v], rows_v, sem).wait()
        pltpu.sync_copy(rows_v, oc_hbm.at[pl.ds(base, bw)])
        pltpu.sync_copy(tpi_hbm.at[pl.ds(base, bw)], idx_v)
        pltpu.async_copy(p_hbm.at[idx_v], rows_v, sem).wait()
        pltpu.sync_copy(rows_v, op_hbm.at[pl.ds(base, bw)])

    return k(cemb, pemb, tci, tpi)


def _head_call(pair, w1T, b1, w2p, b2p, br):
    grid = (NPAIR // br,)

    def body(p_ref, w1_ref, b1_ref, w2_ref, b2_ref, out_ref):
        hdn = jnp.maximum(_mm(p_ref[...], w1_ref[...]) + b1_ref[...], 0.0)
        lg = _mm(hdn, w2_ref[...]) + b2_ref[...]
        out_ref[...] = jax.nn.sigmoid(lg)

    return pl.pallas_call(
        body, grid=grid,
        in_specs=[pl.BlockSpec((br, 2 * HID), lambda b: (b, 0)),
                  pl.BlockSpec((2 * HID, HID), lambda b: (0, 0)),
                  pl.BlockSpec((1, HID), lambda b: (0, 0)),
                  pl.BlockSpec((HID, HID), lambda b: (0, 0)),
                  pl.BlockSpec((1, HID), lambda b: (0, 0))],
        out_specs=pl.BlockSpec((br, HID), lambda b: (b, 0)),
        out_shape=jax.ShapeDtypeStruct((NPAIR, HID), F32),
    )(pair, w1T, b1, w2p, b2p)


def _fold(lp):
    bd = jax.scipy.linalg.block_diag
    f = {}
    f['akc'] = bd(*[lp['a_exp'][h] * (lp['p_exp'][h] * SCALE)
                    for h in range(NHEAD)])
    f['avc'] = bd(*[lp['m_exp'][h] for h in range(NHEAD)])
    f['akp'] = bd(*[lp['a_imp'][h] * (lp['p_imp'][h] * SCALE)
                    for h in range(NHEAD)])
    f['avp'] = bd(*[lp['m_imp'][h] for h in range(NHEAD)])
    for nt in ('c', 'p'):
        for pr in ('q', 'k', 'v', 'o'):
            f[pr + nt + 'T'] = lp[pr + '_' + nt + '_w'].T
            f['b' + pr + nt] = lp[pr + '_' + nt + '_b'][None]
        f['sk' + nt] = lp['skip_' + nt].reshape(1, 1)
    return f


def kernel(edge_exports, edge_imports, target_country_idx, target_product_idx,
           params):
    p = params
    flat_e, flat_i = _flat_call(edge_exports.astype(jnp.int32),
                                edge_imports.astype(jnp.int32))
    zrow = jnp.zeros((SLICE,), F32)
    ce_raw, ci_raw = _counts_call(flat_e, flat_i, zrow)
    c_exp = ce_raw.reshape(TT, NPN, NCN)
    c_imp = ci_raw.reshape(TT, NCN, NPN)

    x_c = p['country_emb'][None]
    x_p = p['product_emb'][None]
    for lp in p['layers']:
        f = _fold(lp)
        q_c, kr_c, vr_c = _proj_call(x_c, f['qcT'], f['bqc'], f['kcT'],
                                     f['bkc'], f['akc'], f['vcT'], f['bvc'],
                                     f['avc'], bn=NCN)
        q_p, kr_p, vr_p = _proj_call(x_p, f['qpT'], f['bqp'], f['kpT'],
                                     f['bkp'], f['akp'], f['vpT'], f['bvp'],
                                     f['avp'], bn=512)
        x_p_new = _msg_call(q_p, kr_c, vr_c, c_exp, x_p, f['opT'], f['bop'],
                            f['skp'], bd=512, cs=NCN)
        x_c_new = _msg_call(q_c, kr_p, vr_p, c_imp, x_c, f['ocT'], f['boc'],
                            f['skc'], bd=NCN, cs=2048)
        x_c, x_p = x_c_new, x_p_new

    cseq = jnp.transpose(x_c, (1, 0, 2))
    pseq = jnp.transpose(x_p, (1, 0, 2))
    seq = jnp.concatenate([cseq, pseq], axis=0)
    g = p['gru']
    hfin = _gru_call(seq, g['w_ih'].T, g['w_hh'].T, g['b_ih'][None],
                     g['b_hh'][None], br=768)
    c_emb, p_emb = hfin[:NCN], hfin[NCN:]

    crow, prow = _pair_gather(c_emb, p_emb, target_country_idx.astype(jnp.int32),
                              target_product_idx.astype(jnp.int32))
    pair = jnp.concatenate([crow, prow], axis=1)
    w2p = jnp.zeros((HID, HID), F32).at[:, 0].set(p['lp_w2'][0])
    b2p = jnp.broadcast_to(p['lp_b2'][None], (1, HID))
    out = _head_call(pair, p['lp_w1'].T, p['lp_b1'][None], w2p, b2p, br=1024)
    return out[:, :1]


# speedup vs baseline: 161.1413x; 1.1422x over previous
import functools
import math

import jax
import jax.numpy as jnp
from jax import lax
from jax.experimental import pallas as pl
from jax.experimental.pallas import tpu as pltpu
from jax.experimental.pallas import tpu_sc as plsc

NCN = 256
NPN = 8192
HID = 128
NHEAD = 4
DHD = HID // NHEAD
TT = 8
NE = 100000
NPAIR = 4096
F32 = jnp.float32
SCALE = 1.0 / math.sqrt(DHD)

_HIGH = jax.lax.Precision.DEFAULT

NW = 32
SLICE = 65536
CEDGE = 10000


def _mm(a, b):
    return lax.dot_general(a, b, (((1,), (0,)), ((), ())),
                           preferred_element_type=F32, precision=_HIGH)


def _mm_t(a, b):
    return lax.dot_general(a, b, (((1,), (1,)), ((), ())),
                           preferred_element_type=F32, precision=_HIGH)


def _flat_call(ee, ei):
    def body(ee_ref, ei_ref, fe_ref, fi_ref):
        fe_ref[0] = ee_ref[0, 1:2, :] * NCN + ee_ref[0, 0:1, :]
        fi_ref[0] = ei_ref[0, 1:2, :] * NPN + ei_ref[0, 0:1, :]
    fe, fi = pl.pallas_call(
        body, grid=(TT,),
        in_specs=[pl.BlockSpec((1, 2, NE), lambda t: (t, 0, 0))] * 2,
        out_specs=[pl.BlockSpec((1, 1, NE), lambda t: (t, 0, 0))] * 2,
        out_shape=[jax.ShapeDtypeStruct((TT, 1, NE), jnp.int32)] * 2,
    )(ee, ei)
    return fe.reshape(TT, NE), fi.reshape(TT, NE)


def _counts_call(flat_e, flat_i, zrow):
    mesh = plsc.VectorSubcoreMesh(core_axis_name="c", subcore_axis_name="s")

    nchunk = NE // CEDGE
    unroll = 5

    @functools.partial(
        pl.kernel, mesh=mesh,
        compiler_params=pltpu.CompilerParams(needs_layout_passes=False),
        out_type=[jax.ShapeDtypeStruct((TT * NW * SLICE,), F32)] * 2,
        scratch_types=[pltpu.VMEM((CEDGE,), jnp.int32),
                       pltpu.VMEM((CEDGE,), jnp.int32),
                       pltpu.VMEM((SLICE,), F32),
                       pltpu.SemaphoreType.DMA,
                       pltpu.SemaphoreType.DMA,
                       pltpu.SemaphoreType.DMA],
    )
    def k(fe_hbm, fi_hbm, z_hbm, ce_hbm, ci_hbm, buf0, buf1, acc_v,
          sem0, sem1, zsem):
        wid = lax.axis_index("s") * 2 + lax.axis_index("c")
        base = wid * SLICE
        ones = jnp.full((16,), 1.0, F32)
        lim = jnp.uint32(SLICE)

        def scatter_chunk(buf):
            def per_vec(i, _):
                for u in range(unroll):
                    fv = buf[pl.ds((i * unroll + u) * 16, 16)]
                    rel = fv - base
                    m = plsc.bitcast(rel, jnp.uint32) < lim
                    rel = jnp.where(m, rel, 0)
                    plsc.addupdate_scatter(acc_v, [rel], ones, mask=m)
                return 0
            lax.fori_loop(0, (CEDGE // 16) // unroll, per_vec, 0)

        def chunk_at(src_hbm, t, c):
            off = pl.multiple_of(t * NE + c * CEDGE, 8)
            return src_hbm.at[pl.ds(off, CEDGE)]

        for src_hbm, dst_hbm in ((fe_hbm, ce_hbm), (fi_hbm, ci_hbm)):
            def per_t(t, _):
                zcp = pltpu.async_copy(z_hbm, acc_v, zsem)
                cp0 = pltpu.async_copy(chunk_at(src_hbm, t, 0), buf0, sem0)
                zcp.wait()
                cp0.wait()

                def per_g(g, _):
                    cpa = pltpu.async_copy(chunk_at(src_hbm, t, 2 * g + 1),
                                           buf1, sem1)
                    scatter_chunk(buf0)
                    cpa.wait()
                    cpb = pltpu.async_copy(chunk_at(src_hbm, t, 2 * g + 2),
                                           buf0, sem0)
                    scatter_chunk(buf1)
                    cpb.wait()
                    return 0
                lax.fori_loop(0, nchunk // 2 - 1, per_g, 0)
                cpl = pltpu.async_copy(chunk_at(src_hbm, t, nchunk - 1),
                                       buf1, sem1)
                scatter_chunk(buf0)
                cpl.wait()
                scatter_chunk(buf1)
                ooff = pl.multiple_of((t * NW + wid) * SLICE, 8)
                pltpu.sync_copy(acc_v, dst_hbm.at[pl.ds(ooff, SLICE)])
                return 0
            lax.fori_loop(0, TT, per_t, 0)

    return k(flat_e.reshape(TT * NE), flat_i.reshape(TT * NE), zrow)


def _log_call(ce, ci):
    rows, cols = 1024, (TT * NW * SLICE) // 1024
    br = 64

    def body(ce_ref, ci_ref, le_ref, li_ref):
        le_ref[...] = jnp.log(ce_ref[...])
        li_ref[...] = jnp.log(ci_ref[...])

    spec = pl.BlockSpec((br, cols), lambda b: (b, 0))
    le, li = pl.pallas_call(
        body, grid=(rows // br,),
        in_specs=[spec, spec],
        out_specs=[spec, spec],
        out_shape=[jax.ShapeDtypeStruct((rows, cols), F32)] * 2,
    )(ce.reshape(rows, cols), ci.reshape(rows, cols))
    return le, li


def _proj_call(x, wqT, bq, wkT, bk, ak, wvT, bv, av, bn):
    ttx, n, _ = x.shape
    grid = (ttx, n // bn)

    def body(x_ref, wq_ref, bq_ref, wk_ref, bk_ref, ak_ref, wv_ref, bv_ref,
             av_ref, q_ref, kr_ref, vr_ref):
        xb = x_ref[0]
        q_ref[0] = _mm(xb, wq_ref[...]) + bq_ref[...]
        kb = _mm(xb, wk_ref[...]) + bk_ref[...]
        kr_ref[0] = _mm(kb, ak_ref[...])
        vb = _mm(xb, wv_ref[...]) + bv_ref[...]
        vr_ref[0] = _mm(vb, av_ref[...])

    wspec = pl.BlockSpec((HID, HID), lambda t, b: (0, 0))
    bspec = pl.BlockSpec((1, HID), lambda t, b: (0, 0))
    xspec = pl.BlockSpec((1, bn, HID), lambda t, b: (t, b, 0))
    return pl.pallas_call(
        body, grid=grid,
        in_specs=[xspec, wspec, bspec, wspec, bspec, wspec, wspec, bspec, wspec],
        out_specs=[xspec] * 3,
        out_shape=[jax.ShapeDtypeStruct((ttx, n, HID), F32)] * 3,
    )(x, wqT, bq, wkT, bk, ak, wvT, bv, av)


def _msg_call(qd, krs, vrs, C, xin, owT, ob, skip, bd, cs):
    tq = qd.shape[0]
    tx = xin.shape[0]
    nd, ns = C.shape[1], C.shape[2]
    grid = (TT, nd // bd, ns // cs)
    nch = ns // cs

    dmap = ((lambda t, b, c: (t, b, 0)) if tq > 1 else
            (lambda t, b, c: (0, b, 0)))
    smap = ((lambda t, b, c: (t, c, 0)) if tq > 1 else
            (lambda t, b, c: (0, c, 0)))
    xmap = ((lambda t, b, c: (t, b, 0)) if tx > 1 else
            (lambda t, b, c: (0, b, 0)))

    def body(q_ref, kr_ref, vr_ref, c_ref, x_ref, ow_ref, ob_ref, sk_ref,
             out_ref, acc_ref, den_ref):
        ci = pl.program_id(2)

        @pl.when(ci == 0)
        def _zero():
            acc_ref[...] = jnp.zeros_like(acc_ref)
            den_ref[...] = jnp.zeros_like(den_ref)

        q = q_ref[0]
        kr = kr_ref[0]
        vr = vr_ref[0]
        lc = c_ref[0]
        ones_col = jnp.ones((vr.shape[0], 1), F32)
        for h in range(NHEAD):
            sl = slice(h * DHD, (h + 1) * DHD)
            al = _mm_t(q[:, sl], kr[:, sl])
            w = jnp.exp(al + lc)
            vx = jnp.concatenate([vr[:, sl], ones_col], axis=1)
            r = _mm(w, vx)
            acc_ref[:, sl] += r[:, :DHD]
            den_ref[:, h:h + 1] += r[:, DHD:DHD + 1]

        @pl.when(ci == nch - 1)
        def _fin():
            agg = acc_ref[...]
            den = den_ref[...]
            parts = [agg[:, h * DHD:(h + 1) * DHD] / (den[:, h:h + 1] + 1e-16)
                     for h in range(NHEAD)]
            nag = jnp.concatenate(parts, axis=1)
            o = _mm(jax.nn.gelu(nag), ow_ref[...]) + ob_ref[...]
            bt = jax.nn.sigmoid(sk_ref[0, 0])
            out_ref[0] = bt * o + (1.0 - bt) * x_ref[0]

    return pl.pallas_call(
        body, grid=grid,
        in_specs=[
            pl.BlockSpec((1, bd, HID), dmap),
            pl.BlockSpec((1, cs, HID), smap),
            pl.BlockSpec((1, cs, HID), smap),
            pl.BlockSpec((1, bd, cs), lambda t, b, c: (t, b, c)),
            pl.BlockSpec((1, bd, HID), xmap),
            pl.BlockSpec((HID, HID), lambda t, b, c: (0, 0)),
            pl.BlockSpec((1, HID), lambda t, b, c: (0, 0)),
            pl.BlockSpec((1, 1), lambda t, b, c: (0, 0)),
        ],
        out_specs=pl.BlockSpec((1, bd, HID), lambda t, b, c: (t, b, 0)),
        out_shape=jax.ShapeDtypeStruct((TT, nd, HID), F32),
        scratch_shapes=[pltpu.VMEM((bd, HID), F32),
                        pltpu.VMEM((bd, NHEAD), F32)],
    )(qd, krs, vrs, C, xin, owT, ob, skip)


def _gru_call(seq, wihT, whhT, bih, bhh, br):
    nr = seq.shape[0]
    grid = (nr // br,)

    def body(s_ref, wih_ref, whh_ref, bih_ref, bhh_ref, out_ref):
        h = jnp.zeros((br, HID), F32)
        for t in range(TT):
            x = s_ref[:, t, :]
            gi = _mm(x, wih_ref[...]) + bih_ref[...]
            gh = _mm(h, whh_ref[...]) + bhh_ref[...]
            r = jax.nn.sigmoid(gi[:, :HID] + gh[:, :HID])
            z = jax.nn.sigmoid(gi[:, HID:2 * HID] + gh[:, HID:2 * HID])
            n = jnp.tanh(gi[:, 2 * HID:] + r * gh[:, 2 * HID:])
            h = (1.0 - z) * n + z * h
        out_ref[...] = h

    return pl.pallas_call(
        body, grid=grid,
        in_specs=[pl.BlockSpec((br, TT, HID), lambda b: (b, 0, 0)),
                  pl.BlockSpec((HID, 3 * HID), lambda b: (0, 0)),
                  pl.BlockSpec((HID, 3 * HID), lambda b: (0, 0)),
                  pl.BlockSpec((1, 3 * HID), lambda b: (0, 0)),
                  pl.BlockSpec((1, 3 * HID), lambda b: (0, 0))],
        out_specs=pl.BlockSpec((br, HID), lambda b: (b, 0)),
        out_shape=jax.ShapeDtypeStruct((nr, HID), F32),
    )(seq, wihT, whhT, bih, bhh)


def _pair_gather(cemb, pemb, tci, tpi):
    mesh = plsc.VectorSubcoreMesh(core_axis_name="c", subcore_axis_name="s")
    bw = NPAIR // NW

    @functools.partial(
        pl.kernel, mesh=mesh,
        compiler_params=pltpu.CompilerParams(needs_layout_passes=False),
        out_type=[jax.ShapeDtypeStruct((NPAIR, HID), F32)] * 2,
        scratch_types=[pltpu.VMEM((bw,), jnp.int32),
                       pltpu.VMEM((bw, HID), F32),
                       pltpu.SemaphoreType.DMA],
    )
    def k(c_hbm, p_hbm, tci_hbm, tpi_hbm, oc_hbm, op_hbm, idx_v, rows_v, sem):
        wid = lax.axis_index("s") * 2 + lax.axis_index("c")
        base = pl.multiple_of(wid * bw, 8)
        pltpu.sync_copy(tci_hbm.at[pl.ds(base, bw)], idx_v)
        pltpu.async_copy(c_hbm.at[idx_v], rows_v, sem).wait()
        pltpu.sync_copy(rows_v, oc_hbm.at[pl.ds(base, bw)])
        pltpu.sync_copy(tpi_hbm.at[pl.ds(base, bw)], idx_v)
        pltpu.async_copy(p_hbm.at[idx_v], rows_v, sem).wait()
        pltpu.sync_copy(rows_v, op_hbm.at[pl.ds(base, bw)])

    return k(cemb, pemb, tci, tpi)


def _head_call(pair, w1T, b1, w2p, b2p, br):
    grid = (NPAIR // br,)

    def body(p_ref, w1_ref, b1_ref, w2_ref, b2_ref, out_ref):
        hdn = jnp.maximum(_mm(p_ref[...], w1_ref[...]) + b1_ref[...], 0.0)
        lg = _mm(hdn, w2_ref[...]) + b2_ref[...]
        out_ref[...] = jax.nn.sigmoid(lg)

    return pl.pallas_call(
        body, grid=grid,
        in_specs=[pl.BlockSpec((br, 2 * HID), lambda b: (b, 0)),
                  pl.BlockSpec((2 * HID, HID), lambda b: (0, 0)),
                  pl.BlockSpec((1, HID), lambda b: (0, 0)),
                  pl.BlockSpec((HID, HID), lambda b: (0, 0)),
                  pl.BlockSpec((1, HID), lambda b: (0, 0))],
        out_specs=pl.BlockSpec((br, HID), lambda b: (b, 0)),
        out_shape=jax.ShapeDtypeStruct((NPAIR, HID), F32),
    )(pair, w1T, b1, w2p, b2p)


def _fold(lp):
    bd = jax.scipy.linalg.block_diag
    f = {}
    f['akc'] = bd(*[lp['a_exp'][h] * (lp['p_exp'][h] * SCALE)
                    for h in range(NHEAD)])
    f['avc'] = bd(*[lp['m_exp'][h] for h in range(NHEAD)])
    f['akp'] = bd(*[lp['a_imp'][h] * (lp['p_imp'][h] * SCALE)
                    for h in range(NHEAD)])
    f['avp'] = bd(*[lp['m_imp'][h] for h in range(NHEAD)])
    for nt in ('c', 'p'):
        for pr in ('q', 'k', 'v', 'o'):
            f[pr + nt + 'T'] = lp[pr + '_' + nt + '_w'].T
            f['b' + pr + nt] = lp[pr + '_' + nt + '_b'][None]
        f['sk' + nt] = lp['skip_' + nt].reshape(1, 1)
    return f


def kernel(edge_exports, edge_imports, target_country_idx, target_product_idx,
           params):
    p = params
    flat_e, flat_i = _flat_call(edge_exports.astype(jnp.int32),
                                edge_imports.astype(jnp.int32))
    zrow = jnp.zeros((SLICE,), F32)
    ce_raw, ci_raw = _counts_call(flat_e, flat_i, zrow)
    lce, lci = _log_call(ce_raw, ci_raw)
    c_exp = lce.reshape(TT, NPN, NCN)
    c_imp = lci.reshape(TT, NCN, NPN)

    x_c = p['country_emb'][None]
    x_p = p['product_emb'][None]
    for lp in p['layers']:
        f = _fold(lp)
        q_c, kr_c, vr_c = _proj_call(x_c, f['qcT'], f['bqc'], f['kcT'],
                                     f['bkc'], f['akc'], f['vcT'], f['bvc'],
                                     f['avc'], bn=NCN)
        q_p, kr_p, vr_p = _proj_call(x_p, f['qpT'], f['bqp'], f['kpT'],
                                     f['bkp'], f['akp'], f['vpT'], f['bvp'],
                                     f['avp'], bn=512)
        x_p_new = _msg_call(q_p, kr_c, vr_c, c_exp, x_p, f['opT'], f['bop'],
                            f['skp'], bd=512, cs=NCN)
        x_c_new = _msg_call(q_c, kr_p, vr_p, c_imp, x_c, f['ocT'], f['boc'],
                            f['skc'], bd=NCN, cs=2048)
        x_c, x_p = x_c_new, x_p_new

    cseq = jnp.transpose(x_c, (1, 0, 2))
    pseq = jnp.transpose(x_p, (1, 0, 2))
    seq = jnp.concatenate([cseq, pseq], axis=0)
    g = p['gru']
    hfin = _gru_call(seq, g['w_ih'].T, g['w_hh'].T, g['b_ih'][None],
                     g['b_hh'][None], br=768)
    c_emb, p_emb = hfin[:NCN], hfin[NCN:]

    crow, prow = _pair_gather(c_emb, p_emb, target_country_idx.astype(jnp.int32),
                              target_product_idx.astype(jnp.int32))
    pair = jnp.concatenate([crow, prow], axis=1)
    w2p = jnp.zeros((HID, HID), F32).at[:, 0].set(p['lp_w2'][0])
    b2p = jnp.broadcast_to(p['lp_b2'][None], (1, HID))
    out = _head_call(pair, p['lp_w1'].T, p['lp_b1'][None], w2p, b2p, br=1024)
    return out[:, :1]


# speedup vs baseline: 173.1914x; 1.0748x over previous
import functools
import math

import jax
import jax.numpy as jnp
from jax import lax
from jax.experimental import pallas as pl
from jax.experimental.pallas import tpu as pltpu
from jax.experimental.pallas import tpu_sc as plsc

NCN = 256
NPN = 8192
HID = 128
NHEAD = 4
DHD = HID // NHEAD
TT = 8
NE = 100000
NPAIR = 4096
F32 = jnp.float32
SCALE = 1.0 / math.sqrt(DHD)

_HIGH = jax.lax.Precision.DEFAULT

NW = 32
SLICE = 65536
CEDGE = 10000


def _mm(a, b):
    return lax.dot_general(a, b, (((1,), (0,)), ((), ())),
                           preferred_element_type=F32, precision=_HIGH)


def _mm_t(a, b):
    return lax.dot_general(a, b, (((1,), (1,)), ((), ())),
                           preferred_element_type=F32, precision=_HIGH)


def _flat_call(ee, ei):
    def body(ee_ref, ei_ref, fe_ref, fi_ref):
        fe_ref[0] = ee_ref[0, 1:2, :] * NCN + ee_ref[0, 0:1, :]
        fi_ref[0] = ei_ref[0, 1:2, :] * NPN + ei_ref[0, 0:1, :]
    fe, fi = pl.pallas_call(
        body, grid=(TT,),
        in_specs=[pl.BlockSpec((1, 2, NE), lambda t: (t, 0, 0))] * 2,
        out_specs=[pl.BlockSpec((1, 1, NE), lambda t: (t, 0, 0))] * 2,
        out_shape=[jax.ShapeDtypeStruct((TT, 1, NE), jnp.int32)] * 2,
    )(ee, ei)
    return fe.reshape(TT, NE), fi.reshape(TT, NE)


def _counts_call(flat, zrow):
    mesh = plsc.VectorSubcoreMesh(core_axis_name="c", subcore_axis_name="s")
    nchunk = NE // CEDGE
    unroll = 5

    @functools.partial(
        pl.kernel, mesh=mesh,
        compiler_params=pltpu.CompilerParams(needs_layout_passes=False),
        out_type=jax.ShapeDtypeStruct((TT * NW * SLICE,), F32),
        scratch_types=[pltpu.VMEM((CEDGE,), jnp.int32),
                       pltpu.VMEM((CEDGE,), jnp.int32),
                       pltpu.VMEM((SLICE,), F32),
                       pltpu.SemaphoreType.DMA,
                       pltpu.SemaphoreType.DMA,
                       pltpu.SemaphoreType.DMA],
    )
    def k(src_hbm, z_hbm, dst_hbm, buf0, buf1, acc_v, sem0, sem1, zsem):
        wid = lax.axis_index("s") * 2 + lax.axis_index("c")
        base = wid * SLICE
        ones = jnp.full((16,), 1.0, F32)
        lim = jnp.uint32(SLICE)

        def scatter_chunk(buf):
            def per_vec(i, _):
                for u in range(unroll):
                    fv = buf[pl.ds((i * unroll + u) * 16, 16)]
                    rel = fv - base
                    m = plsc.bitcast(rel, jnp.uint32) < lim
                    rel = jnp.where(m, rel, 0)
                    plsc.addupdate_scatter(acc_v, [rel], ones, mask=m)
                return 0
            lax.fori_loop(0, (CEDGE // 16) // unroll, per_vec, 0)

        def chunk_at(t, c):
            off = pl.multiple_of(t * NE + c * CEDGE, 8)
            return src_hbm.at[pl.ds(off, CEDGE)]

        def per_t(t, _):
            zcp = pltpu.async_copy(z_hbm, acc_v, zsem)
            cp0 = pltpu.async_copy(chunk_at(t, 0), buf0, sem0)
            zcp.wait()
            cp0.wait()

            def per_g(g, _):
                cpa = pltpu.async_copy(chunk_at(t, 2 * g + 1), buf1, sem1)
                scatter_chunk(buf0)
                cpa.wait()
                cpb = pltpu.async_copy(chunk_at(t, 2 * g + 2), buf0, sem0)
                scatter_chunk(buf1)
                cpb.wait()
                return 0
            lax.fori_loop(0, nchunk // 2 - 1, per_g, 0)
            cpl = pltpu.async_copy(chunk_at(t, nchunk - 1), buf1, sem1)
            scatter_chunk(buf0)
            cpl.wait()
            scatter_chunk(buf1)
            ooff = pl.multiple_of((t * NW + wid) * SLICE, 8)
            pltpu.sync_copy(acc_v, dst_hbm.at[pl.ds(ooff, SLICE)])
            return 0
        lax.fori_loop(0, TT, per_t, 0)

    return k(flat.reshape(TT * NE), zrow)


def _log_call(c_raw):
    rows, cols = 1024, (TT * NW * SLICE) // 1024
    br = 64

    def body(c_ref, l_ref):
        l_ref[...] = jnp.log(c_ref[...])

    spec = pl.BlockSpec((br, cols), lambda b: (b, 0))
    return pl.pallas_call(
        body, grid=(rows // br,),
        in_specs=[spec],
        out_specs=spec,
        out_shape=jax.ShapeDtypeStruct((rows, cols), F32),
    )(c_raw.reshape(rows, cols))


def _proj_call(x, wqT, bq, wkT, bk, ak, wvT, bv, av, bn):
    ttx, n, _ = x.shape
    grid = (ttx, n // bn)

    def body(x_ref, wq_ref, bq_ref, wk_ref, bk_ref, ak_ref, wv_ref, bv_ref,
             av_ref, q_ref, kr_ref, vr_ref):
        xb = x_ref[0]
        q_ref[0] = _mm(xb, wq_ref[...]) + bq_ref[...]
        kb = _mm(xb, wk_ref[...]) + bk_ref[...]
        kr_ref[0] = _mm(kb, ak_ref[...])
        vb = _mm(xb, wv_ref[...]) + bv_ref[...]
        vr_ref[0] = _mm(vb, av_ref[...])

    wspec = pl.BlockSpec((HID, HID), lambda t, b: (0, 0))
    bspec = pl.BlockSpec((1, HID), lambda t, b: (0, 0))
    xspec = pl.BlockSpec((1, bn, HID), lambda t, b: (t, b, 0))
    return pl.pallas_call(
        body, grid=grid,
        in_specs=[xspec, wspec, bspec, wspec, bspec, wspec, wspec, bspec, wspec],
        out_specs=[xspec] * 3,
        out_shape=[jax.ShapeDtypeStruct((ttx, n, HID), F32)] * 3,
    )(x, wqT, bq, wkT, bk, ak, wvT, bv, av)


def _msg_call(qd, krs, vrs, C, xin, owT, ob, skip, bd, cs):
    tq = qd.shape[0]
    tx = xin.shape[0]
    nd, ns = C.shape[1], C.shape[2]
    grid = (TT, nd // bd, ns // cs)
    nch = ns // cs

    dmap = ((lambda t, b, c: (t, b, 0)) if tq > 1 else
            (lambda t, b, c: (0, b, 0)))
    smap = ((lambda t, b, c: (t, c, 0)) if tq > 1 else
            (lambda t, b, c: (0, c, 0)))
    xmap = ((lambda t, b, c: (t, b, 0)) if tx > 1 else
            (lambda t, b, c: (0, b, 0)))

    def body(q_ref, kr_ref, vr_ref, c_ref, x_ref, ow_ref, ob_ref, sk_ref,
             out_ref, acc_ref, den_ref):
        ci = pl.program_id(2)

        @pl.when(ci == 0)
        def _zero():
            acc_ref[...] = jnp.zeros_like(acc_ref)
            den_ref[...] = jnp.zeros_like(den_ref)

        q = q_ref[0]
        kr = kr_ref[0]
        vr = vr_ref[0]
        lc = c_ref[0]
        ones_col = jnp.ones((vr.shape[0], 1), F32)
        for h in range(NHEAD):
            sl = slice(h * DHD, (h + 1) * DHD)
            al = _mm_t(q[:, sl], kr[:, sl])
            w = jnp.exp(al + lc)
            vx = jnp.concatenate([vr[:, sl], ones_col], axis=1)
            r = _mm(w, vx)
            acc_ref[:, sl] += r[:, :DHD]
            den_ref[:, h:h + 1] += r[:, DHD:DHD + 1]

        @pl.when(ci == nch - 1)
        def _fin():
            agg = acc_ref[...]
            den = den_ref[...]
            parts = [agg[:, h * DHD:(h + 1) * DHD] / (den[:, h:h + 1] + 1e-16)
                     for h in range(NHEAD)]
            nag = jnp.concatenate(parts, axis=1)
            o = _mm(jax.nn.gelu(nag), ow_ref[...]) + ob_ref[...]
            bt = jax.nn.sigmoid(sk_ref[0, 0])
            out_ref[0] = bt * o + (1.0 - bt) * x_ref[0]

    return pl.pallas_call(
        body, grid=grid,
        in_specs=[
            pl.BlockSpec((1, bd, HID), dmap),
            pl.BlockSpec((1, cs, HID), smap),
            pl.BlockSpec((1, cs, HID), smap),
            pl.BlockSpec((1, bd, cs), lambda t, b, c: (t, b, c)),
            pl.BlockSpec((1, bd, HID), xmap),
            pl.BlockSpec((HID, HID), lambda t, b, c: (0, 0)),
            pl.BlockSpec((1, HID), lambda t, b, c: (0, 0)),
            pl.BlockSpec((1, 1), lambda t, b, c: (0, 0)),
        ],
        out_specs=pl.BlockSpec((1, bd, HID), lambda t, b, c: (t, b, 0)),
        out_shape=jax.ShapeDtypeStruct((TT, nd, HID), F32),
        scratch_shapes=[pltpu.VMEM((bd, HID), F32),
                        pltpu.VMEM((bd, NHEAD), F32)],
    )(qd, krs, vrs, C, xin, owT, ob, skip)


def _gru_call(seq, wihT, whhT, bih, bhh, br):
    nr = seq.shape[0]
    grid = (nr // br,)

    def body(s_ref, wih_ref, whh_ref, bih_ref, bhh_ref, out_ref):
        h = jnp.zeros((br, HID), F32)
        for t in range(TT):
            x = s_ref[:, t, :]
            gi = _mm(x, wih_ref[...]) + bih_ref[...]
            gh = _mm(h, whh_ref[...]) + bhh_ref[...]
            r = jax.nn.sigmoid(gi[:, :HID] + gh[:, :HID])
            z = jax.nn.sigmoid(gi[:, HID:2 * HID] + gh[:, HID:2 * HID])
            n = jnp.tanh(gi[:, 2 * HID:] + r * gh[:, 2 * HID:])
            h = (1.0 - z) * n + z * h
        out_ref[...] = h

    return pl.pallas_call(
        body, grid=grid,
        in_specs=[pl.BlockSpec((br, TT, HID), lambda b: (b, 0, 0)),
                  pl.BlockSpec((HID, 3 * HID), lambda b: (0, 0)),
                  pl.BlockSpec((HID, 3 * HID), lambda b: (0, 0)),
                  pl.BlockSpec((1, 3 * HID), lambda b: (0, 0)),
                  pl.BlockSpec((1, 3 * HID), lambda b: (0, 0))],
        out_specs=pl.BlockSpec((br, HID), lambda b: (b, 0)),
        out_shape=jax.ShapeDtypeStruct((nr, HID), F32),
    )(seq, wihT, whhT, bih, bhh)


def _pair_gather(cemb, pemb, tci, tpi):
    mesh = plsc.VectorSubcoreMesh(core_axis_name="c", subcore_axis_name="s")
    bw = NPAIR // NW

    @functools.partial(
        pl.kernel, mesh=mesh,
        compiler_params=pltpu.CompilerParams(needs_layout_passes=False),
        out_type=[jax.ShapeDtypeStruct((NPAIR, HID), F32)] * 2,
        scratch_types=[pltpu.VMEM((bw,), jnp.int32),
                       pltpu.VMEM((bw, HID), F32),
                       pltpu.SemaphoreType.DMA],
    )
    def k(c_hbm, p_hbm, tci_hbm, tpi_hbm, oc_hbm, op_hbm, idx_v, rows_v, sem):
        wid = lax.axis_index("s") * 2 + lax.axis_index("c")
        base = pl.multiple_of(wid * bw, 8)
        pltpu.sync_copy(tci_hbm.at[pl.ds(base, bw)], idx_v)
        pltpu.async_copy(c_hbm.at[idx_v], rows_v, sem).wait()
        pltpu.sync_copy(rows_v, oc_hbm.at[pl.ds(base, bw)])
        pltpu.sync_copy(tpi_hbm.at[pl.ds(base, bw)], idx_v)
        pltpu.async_copy(p_hbm.at[idx_v], rows_v, sem).wait()
        pltpu.sync_copy(rows_v, op_hbm.at[pl.ds(base, bw)])

    return k(cemb, pemb, tci, tpi)


def _head_call(pair, w1T, b1, w2p, b2p, br):
    grid = (NPAIR // br,)

    def body(p_ref, w1_ref, b1_ref, w2_ref, b2_ref, out_ref):
        hdn = jnp.maximum(_mm(p_ref[...], w1_ref[...]) + b1_ref[...], 0.0)
        lg = _mm(hdn, w2_ref[...]) + b2_ref[...]
        out_ref[...] = jax.nn.sigmoid(lg)

    return pl.pallas_call(
        body, grid=grid,
        in_specs=[pl.BlockSpec((br, 2 * HID), lambda b: (b, 0)),
                  pl.BlockSpec((2 * HID, HID), lambda b: (0, 0)),
                  pl.BlockSpec((1, HID), lambda b: (0, 0)),
                  pl.BlockSpec((HID, HID), lambda b: (0, 0)),
                  pl.BlockSpec((1, HID), lambda b: (0, 0))],
        out_specs=pl.BlockSpec((br, HID), lambda b: (b, 0)),
        out_shape=jax.ShapeDtypeStruct((NPAIR, HID), F32),
    )(pair, w1T, b1, w2p, b2p)


def _fold(lp):
    bd = jax.scipy.linalg.block_diag
    f = {}
    f['akc'] = bd(*[lp['a_exp'][h] * (lp['p_exp'][h] * SCALE)
                    for h in range(NHEAD)])
    f['avc'] = bd(*[lp['m_exp'][h] for h in range(NHEAD)])
    f['akp'] = bd(*[lp['a_imp'][h] * (lp['p_imp'][h] * SCALE)
                    for h in range(NHEAD)])
    f['avp'] = bd(*[lp['m_imp'][h] for h in range(NHEAD)])
    for nt in ('c', 'p'):
        for pr in ('q', 'k', 'v', 'o'):
            f[pr + nt + 'T'] = lp[pr + '_' + nt + '_w'].T
            f['b' + pr + nt] = lp[pr + '_' + nt + '_b'][None]
        f['sk' + nt] = lp['skip_' + nt].reshape(1, 1)
    return f


def kernel(edge_exports, edge_imports, target_country_idx, target_product_idx,
           params):
    p = params
    flat_e, flat_i = _flat_call(edge_exports.astype(jnp.int32),
                                edge_imports.astype(jnp.int32))
    zrow = jnp.zeros((SLICE,), F32)
    ce_raw = _counts_call(flat_e, zrow)
    c_exp = _log_call(ce_raw).reshape(TT, NPN, NCN)
    ci_raw = _counts_call(flat_i, zrow)
    c_imp = _log_call(ci_raw).reshape(TT, NCN, NPN)

    x_c = p['country_emb'][None]
    x_p = p['product_emb'][None]
    for lp in p['layers']:
        f = _fold(lp)
        q_c, kr_c, vr_c = _proj_call(x_c, f['qcT'], f['bqc'], f['kcT'],
                                     f['bkc'], f['akc'], f['vcT'], f['bvc'],
                                     f['avc'], bn=NCN)
        q_p, kr_p, vr_p = _proj_call(x_p, f['qpT'], f['bqp'], f['kpT'],
                                     f['bkp'], f['akp'], f['vpT'], f['bvp'],
                                     f['avp'], bn=512)
        x_p_new = _msg_call(q_p, kr_c, vr_c, c_exp, x_p, f['opT'], f['bop'],
                            f['skp'], bd=512, cs=NCN)
        x_c_new = _msg_call(q_c, kr_p, vr_p, c_imp, x_c, f['ocT'], f['boc'],
                            f['skc'], bd=NCN, cs=2048)
        x_c, x_p = x_c_new, x_p_new

    cseq = jnp.transpose(x_c, (1, 0, 2))
    pseq = jnp.transpose(x_p, (1, 0, 2))
    seq = jnp.concatenate([cseq, pseq], axis=0)
    g = p['gru']
    hfin = _gru_call(seq, g['w_ih'].T, g['w_hh'].T, g['b_ih'][None],
                     g['b_hh'][None], br=768)
    c_emb, p_emb = hfin[:NCN], hfin[NCN:]

    crow, prow = _pair_gather(c_emb, p_emb, target_country_idx.astype(jnp.int32),
                              target_product_idx.astype(jnp.int32))
    pair = jnp.concatenate([crow, prow], axis=1)
    w2p = jnp.zeros((HID, HID), F32).at[:, 0].set(p['lp_w2'][0])
    b2p = jnp.broadcast_to(p['lp_b2'][None], (1, HID))
    out = _head_call(pair, p['lp_w1'].T, p['lp_b1'][None], w2p, b2p, br=1024)
    return out[:, :1]
